# baseline (speedup 1.0000x reference)
import functools

import jax
import jax.numpy as jnp
from jax import lax
from jax.experimental import pallas as pl
from jax.experimental.pallas import tpu as pltpu
from jax.experimental.pallas import tpu_sc as plsc

N_NODES = 10000
N_EDGES = 320000
D_FEAT = 128
DIM_H = 128
N_CLASSES = 32
BN_EPS = 1e-5

NC = 2
NS = 16
NW = NC * NS
NG = 8
NH = 2
NEH = 2
HALF = N_NODES // NH
CHUNK = 640
EDGES_PER_EH = N_EDGES // NEH
N_CHUNKS = EDGES_PER_EH // CHUNK
UNROLL = 8
NB_I = 5
NB_M = 4

_mesh = plsc.VectorSubcoreMesh(core_axis_name="c", subcore_axis_name="s")


@functools.partial(
    pl.kernel,
    mesh=_mesh,
    compiler_params=pltpu.CompilerParams(needs_layout_passes=False,
                                         use_tc_tiling_on_sc=False),
    out_type=jax.ShapeDtypeStruct((NEH, NH, HALF, D_FEAT), jnp.float32),
    scratch_types=(
        [pltpu.VMEM((CHUNK,), jnp.int32) for _ in range(NB_I)]
        + [pltpu.VMEM((CHUNK,), jnp.int32) for _ in range(NB_I)]
        + [pltpu.VMEM((CHUNK, 16), jnp.float32) for _ in range(NB_M)]
        + [pltpu.VMEM((HALF + 16, 16), jnp.float32)]
        + [pltpu.SemaphoreType.DMA for _ in range(NB_I + NB_M)]
    ),
)
def _agg_kernel(xr_hbm, src_hbm, dst_hbm, out_hbm, *refs):
    srcs = list(refs[0:NB_I])
    dsts = list(refs[NB_I:2 * NB_I])
    msgs = list(refs[2 * NB_I:2 * NB_I + NB_M])
    acc_v = refs[2 * NB_I + NB_M]
    sis = list(refs[2 * NB_I + NB_M + 1:2 * NB_I + NB_M + 1 + NB_I])
    sgs = list(refs[2 * NB_I + NB_M + 1 + NB_I:])
    cid = lax.axis_index("c")
    sid = lax.axis_index("s")
    w = sid * NC + cid
    g = w % NG
    nh = (w // NG) % NH
    eh = w // (NG * NH)

    zv = jnp.zeros((16,), jnp.float32)

    @plsc.parallel_loop(0, HALF // UNROLL)
    def _zb(i):
        for u in range(UNROLL):
            acc_v[i * UNROLL + u] = zv

    lo = nh * HALF
    lov = jnp.full((16,), lo, jnp.int32)
    iota = lax.iota(jnp.int32, 16)
    gv = jnp.broadcast_to(g, (16,))
    trashv = jnp.full((16,), HALF, jnp.uint32) + plsc.bitcast(iota,
                                                              jnp.uint32)
    ebase = eh * EDGES_PER_EH
    LAST = N_CHUNKS - 1

    def idx_issue(c, b):
        base = ebase + c * CHUNK
        pltpu.async_copy(src_hbm.at[pl.ds(base, CHUNK)], srcs[b], sis[b])
        pltpu.async_copy(dst_hbm.at[pl.ds(base, CHUNK)], dsts[b], sis[b])

    def idx_wait(b):
        pltpu.make_async_copy(src_hbm.at[pl.ds(0, CHUNK)], srcs[b],
                              sis[b]).wait()
        pltpu.make_async_copy(dst_hbm.at[pl.ds(0, CHUNK)], dsts[b],
                              sis[b]).wait()

    def transform(b):
        src_v = srcs[b]

        @plsc.parallel_loop(0, CHUNK // 16, unroll=4)
        def _t(i):
            s16 = src_v[pl.ds(i * 16, 16)]
            src_v[pl.ds(i * 16, 16)] = s16 * NG + gv

    def g_issue(b_i, b_m):
        return pltpu.async_copy(xr_hbm.at[srcs[b_i]], msgs[b_m], sgs[b_m])

    def compute(b_m, b_i):
        msg_v = msgs[b_m]
        dst_v = dsts[b_i]

        @plsc.parallel_loop(0, CHUNK // 16, unroll=4)
        def _edges(j):
            dsts16 = dst_v[pl.ds(j * 16, 16)]
            rows = jnp.minimum(
                plsc.bitcast(dsts16 - lov, jnp.uint32), trashv)
            for u in range(16):
                rowv = plsc.bitcast(jnp.broadcast_to(rows[u], (16,)),
                                    jnp.int32)
                vals = msg_v[j * 16 + u]
                plsc.addupdate_scatter(acc_v, [rowv, iota], vals)

    for b in range(NB_I):
        idx_issue(jnp.int32(b), b)

    def _body(kk, _):
        c0 = kk * NB_I
        for b in range(NB_I):
            idx_wait(b)
        for b in range(NB_M):
            transform(b)
        gh = [g_issue(b, b) for b in range(NB_M)]
        for b in range(NB_I):
            gh[b].wait()
            compute(b % NB_M, b)
            idx_issue(jnp.minimum(c0 + NB_I + b, LAST), b)
            if b + NB_M < NB_I:
                transform(b + NB_M)
                gh.append(g_issue(b + NB_M, (b + NB_M) % NB_M))
        return _

    lax.fori_loop(0, N_CHUNKS // NB_I, _body, None)
    for b in range(NB_I):
        idx_wait(b)

    pltpu.sync_copy(acc_v.at[pl.ds(0, HALF)],
                    out_hbm.at[eh].at[nh].at[:, pl.ds(g * 16, 16)])


def _mlp_body(x_ref, a0_ref, a1_ref, W1_ref, b1_ref, g_ref, be_ref, W2_ref,
              b2_ref, Wl_ref, bl_ref, o_ref):
    h = x_ref[...] + a0_ref[...] + a1_ref[...]
    h = jnp.dot(h, W1_ref[...], preferred_element_type=jnp.float32) + b1_ref[...]
    mu = jnp.mean(h, axis=0, keepdims=True)
    var = jnp.mean(jnp.square(h - mu), axis=0, keepdims=True)
    h = (h - mu) / jnp.sqrt(var + BN_EPS) * g_ref[...] + be_ref[...]
    h = jnp.maximum(h, 0.0)
    h = jnp.dot(h, W2_ref[...], preferred_element_type=jnp.float32) + b2_ref[...]
    h = jnp.maximum(h, 0.0)
    o_ref[...] = (jnp.dot(h, Wl_ref[...], preferred_element_type=jnp.float32)
                  + bl_ref[...])


_mlp_call = pl.pallas_call(
    _mlp_body,
    out_shape=jax.ShapeDtypeStruct((N_NODES, N_CLASSES), jnp.float32),
)


def kernel(x, edge_index, W1, b1, gamma, beta, W2, b2, Wl, bl):
    ei = edge_index.astype(jnp.int32)
    xr = x.reshape(N_NODES * NG, 16)
    parts = _agg_kernel(xr, ei[0], ei[1])
    aggs = parts.reshape(NEH, N_NODES, D_FEAT)
    return _mlp_call(x, aggs[0], aggs[1], W1, b1.reshape(1, -1),
                     gamma.reshape(1, -1), beta.reshape(1, -1), W2,
                     b2.reshape(1, -1), Wl, bl.reshape(1, -1))


# speedup vs baseline: 1.1922x; 1.1922x over previous
import functools

import jax
import jax.numpy as jnp
from jax import lax
from jax.experimental import pallas as pl
from jax.experimental.pallas import tpu as pltpu
from jax.experimental.pallas import tpu_sc as plsc

N_NODES = 10000
N_EDGES = 320000
D_FEAT = 128
DIM_H = 128
N_CLASSES = 32
BN_EPS = 1e-5

NC = 2
NS = 16
NW = NC * NS
NG = 8
NH = 2
NEH = 2
HALF = N_NODES // NH
CHUNK = 640
EDGES_PER_EH = N_EDGES // NEH
N_CHUNKS = EDGES_PER_EH // CHUNK
UNROLL = 8
NB_I = 10
NB_M = 3

_mesh = plsc.VectorSubcoreMesh(core_axis_name="c", subcore_axis_name="s")


@functools.partial(
    pl.kernel,
    mesh=_mesh,
    compiler_params=pltpu.CompilerParams(needs_layout_passes=False,
                                         use_tc_tiling_on_sc=False),
    out_type=jax.ShapeDtypeStruct((NEH, NH, HALF, D_FEAT), jnp.float32),
    scratch_types=(
        [pltpu.VMEM((CHUNK,), jnp.int32) for _ in range(NB_I)]
        + [pltpu.VMEM((CHUNK,), jnp.int32) for _ in range(NB_I)]
        + [pltpu.VMEM((CHUNK, 16), jnp.float32) for _ in range(NB_M)]
        + [pltpu.VMEM((HALF + 16, 16), jnp.float32)]
        + [pltpu.SemaphoreType.DMA for _ in range(NB_I + NB_M)]
    ),
)
def _agg_kernel(xr_hbm, src_hbm, dst_hbm, out_hbm, *refs):
    srcs = list(refs[0:NB_I])
    dsts = list(refs[NB_I:2 * NB_I])
    msgs = list(refs[2 * NB_I:2 * NB_I + NB_M])
    acc_v = refs[2 * NB_I + NB_M]
    sis = list(refs[2 * NB_I + NB_M + 1:2 * NB_I + NB_M + 1 + NB_I])
    sgs = list(refs[2 * NB_I + NB_M + 1 + NB_I:])
    cid = lax.axis_index("c")
    sid = lax.axis_index("s")
    w = sid * NC + cid
    g = w % NG
    nh = (w // NG) % NH
    eh = w // (NG * NH)

    zv = jnp.zeros((16,), jnp.float32)

    @plsc.parallel_loop(0, HALF // UNROLL)
    def _zb(i):
        for u in range(UNROLL):
            acc_v[i * UNROLL + u] = zv

    lo = nh * HALF
    lov = jnp.full((16,), lo, jnp.int32)
    iota = lax.iota(jnp.int32, 16)
    gv = jnp.broadcast_to(g, (16,))
    trashv = jnp.full((16,), HALF, jnp.uint32) + plsc.bitcast(iota,
                                                              jnp.uint32)
    ebase = eh * EDGES_PER_EH
    LAST = N_CHUNKS - 1

    def idx_issue(c, b):
        base = ebase + c * CHUNK
        pltpu.async_copy(src_hbm.at[pl.ds(base, CHUNK)], srcs[b], sis[b])
        pltpu.async_copy(dst_hbm.at[pl.ds(base, CHUNK)], dsts[b], sis[b])

    def idx_wait(b):
        pltpu.make_async_copy(src_hbm.at[pl.ds(0, CHUNK)], srcs[b],
                              sis[b]).wait()
        pltpu.make_async_copy(dst_hbm.at[pl.ds(0, CHUNK)], dsts[b],
                              sis[b]).wait()

    def transform(b):
        src_v = srcs[b]

        @plsc.parallel_loop(0, CHUNK // 16, unroll=4)
        def _t(i):
            s16 = src_v[pl.ds(i * 16, 16)]
            src_v[pl.ds(i * 16, 16)] = s16 * NG + gv

    def g_issue(b_i, b_m):
        return pltpu.async_copy(xr_hbm.at[srcs[b_i]], msgs[b_m], sgs[b_m])

    def compute(b_m, b_i):
        msg_v = msgs[b_m]
        dst_v = dsts[b_i]

        @plsc.parallel_loop(0, CHUNK // 16, unroll=4)
        def _edges(j):
            dsts16 = dst_v[pl.ds(j * 16, 16)]
            rows = jnp.minimum(
                plsc.bitcast(dsts16 - lov, jnp.uint32), trashv)
            for u in range(16):
                rowv = plsc.bitcast(jnp.broadcast_to(rows[u], (16,)),
                                    jnp.int32)
                vals = msg_v[j * 16 + u]
                plsc.addupdate_scatter(acc_v, [rowv, iota], vals)

    for b in range(NB_I):
        idx_issue(jnp.int32(b), b)

    def _body(kk, _):
        c0 = kk * NB_I
        for b in range(NB_I):
            idx_wait(b)
        for b in range(NB_M):
            transform(b)
        gh = [g_issue(b, b) for b in range(NB_M)]
        for b in range(NB_I):
            gh[b].wait()
            compute(b % NB_M, b)
            idx_issue(jnp.minimum(c0 + NB_I + b, LAST), b)
            if b + NB_M < NB_I:
                transform(b + NB_M)
                gh.append(g_issue(b + NB_M, (b + NB_M) % NB_M))
        return _

    lax.fori_loop(0, N_CHUNKS // NB_I, _body, None)
    for b in range(NB_I):
        idx_wait(b)

    pltpu.sync_copy(acc_v.at[pl.ds(0, HALF)],
                    out_hbm.at[eh].at[nh].at[:, pl.ds(g * 16, 16)])


def _mlp_body(x_ref, a0_ref, a1_ref, W1_ref, b1_ref, g_ref, be_ref, W2_ref,
              b2_ref, Wl_ref, bl_ref, o_ref):
    h = x_ref[...] + a0_ref[...] + a1_ref[...]
    h = jnp.dot(h, W1_ref[...], preferred_element_type=jnp.float32) + b1_ref[...]
    mu = jnp.mean(h, axis=0, keepdims=True)
    var = jnp.mean(jnp.square(h - mu), axis=0, keepdims=True)
    h = (h - mu) / jnp.sqrt(var + BN_EPS) * g_ref[...] + be_ref[...]
    h = jnp.maximum(h, 0.0)
    h = jnp.dot(h, W2_ref[...], preferred_element_type=jnp.float32) + b2_ref[...]
    h = jnp.maximum(h, 0.0)
    o_ref[...] = (jnp.dot(h, Wl_ref[...], preferred_element_type=jnp.float32)
                  + bl_ref[...])


_mlp_call = pl.pallas_call(
    _mlp_body,
    out_shape=jax.ShapeDtypeStruct((N_NODES, N_CLASSES), jnp.float32),
)


def kernel(x, edge_index, W1, b1, gamma, beta, W2, b2, Wl, bl):
    ei = edge_index.astype(jnp.int32)
    xr = x.reshape(N_NODES * NG, 16)
    parts = _agg_kernel(xr, ei[0], ei[1])
    aggs = parts.reshape(NEH, N_NODES, D_FEAT)
    return _mlp_call(x, aggs[0], aggs[1], W1, b1.reshape(1, -1),
                     gamma.reshape(1, -1), beta.reshape(1, -1), W2,
                     b2.reshape(1, -1), Wl, bl.reshape(1, -1))
